# baseline (speedup 1.0000x reference)
import functools

import jax
import jax.numpy as jnp
from jax import lax
from jax.experimental import pallas as pl
from jax.experimental.pallas import tpu as pltpu
from jax.experimental.pallas import tpu_sc as plsc

N = 10000
E = 160000
D = 256
DH = 128
NC, NS = 2, 16
NPAD = 10240
ROWS_PER_TILE = NPAD // NS
EPAD = 163840
B = 128
PADROW = N


def _mesh():
    return plsc.VectorSubcoreMesh(
        core_axis_name="c", subcore_axis_name="s", num_cores=NC, num_subcores=NS
    )


_A_NB = EPAD // (NC * NS) // B


def _deg_body(dst_hbm, zeros_hbm, ones_hbm, cnt_hbm, deg_sh, dst_all, ones_v, sem):
    c = lax.axis_index("c")
    s = lax.axis_index("s")
    wid = s * NC + c
    pltpu.async_copy(dst_hbm.at[pl.ds(wid * _A_NB, _A_NB)], dst_all, sem).wait()
    pltpu.async_copy(ones_hbm, ones_v, sem).wait()
    pltpu.async_copy(
        zeros_hbm, deg_sh.at[pl.ds(s * ROWS_PER_TILE, ROWS_PER_TILE)], sem
    ).wait()
    plsc.subcore_barrier()

    def step(j, carry):
        pltpu.sync_copy(ones_v, deg_sh.at[dst_all.at[j]], add=True)
        return carry

    lax.fori_loop(0, _A_NB, step, 0)
    plsc.subcore_barrier()
    pltpu.async_copy(
        deg_sh.at[pl.ds(s * ROWS_PER_TILE, ROWS_PER_TILE)],
        cnt_hbm.at[pl.ds(c * NPAD + s * ROWS_PER_TILE, ROWS_PER_TILE)],
        sem,
    ).wait()


def _deg_call(dst_pad, zeros128, ones128):
    f = pl.kernel(
        _deg_body,
        out_type=jax.ShapeDtypeStruct((NC * NPAD, DH), jnp.float32),
        mesh=_mesh(),
        scratch_types=[
            pltpu.VMEM_SHARED((NPAD, DH), jnp.float32),
            pltpu.VMEM((_A_NB, B), jnp.int32),
            pltpu.VMEM((B, DH), jnp.float32),
            pltpu.SemaphoreType.DMA,
        ],
        name="gcn_deg_sc",
    )
    return f(dst_pad, zeros128, ones128)


_RB = 80


def _h_body(x_ref, w_ref, h_ref):
    h_ref[...] = jnp.dot(x_ref[...], w_ref[...], preferred_element_type=jnp.float32)


def _h_call(x, w):
    return pl.pallas_call(
        _h_body,
        grid=(NC, N // _RB),
        in_specs=[
            pl.BlockSpec((_RB, D), lambda c, i: (i, 0)),
            pl.BlockSpec((D, DH), lambda c, i: (0, c)),
        ],
        out_specs=pl.BlockSpec((_RB, DH), lambda c, i: (c * (NPAD // _RB) + i, 0)),
        out_shape=jax.ShapeDtypeStruct((NC * NPAD, DH), jnp.float32),
        name="gcn_h_tc",
    )(x, w)


def _h2_body(h_ref, cnt0_ref, cnt1_ref, h2_ref, dinv_ref):
    cnt = cnt0_ref[:, 0] + cnt1_ref[:, 0] + 1.0
    dv = lax.rsqrt(cnt)
    h2_ref[...] = h_ref[...] * dv[:, None]
    dinv_ref[...] = dv[:, None]


def _h2_call(h, cnt):
    return pl.pallas_call(
        _h2_body,
        grid=(NC, N // _RB),
        in_specs=[
            pl.BlockSpec((_RB, DH), lambda c, i: (c * (NPAD // _RB) + i, 0)),
            pl.BlockSpec((_RB, DH), lambda c, i: (i, 0)),
            pl.BlockSpec((_RB, DH), lambda c, i: (NPAD // _RB + i, 0)),
        ],
        out_specs=[
            pl.BlockSpec((_RB, DH), lambda c, i: (c * (NPAD // _RB) + i, 0)),
            pl.BlockSpec((_RB, 1), lambda c, i: (i, 0)),
        ],
        out_shape=[
            jax.ShapeDtypeStruct((NC * NPAD, DH), jnp.float32),
            jax.ShapeDtypeStruct((N, 1), jnp.float32),
        ],
        name="gcn_h2_tc",
    )(h, cnt, cnt)


_C_NB = EPAD // NS // B
_C_PH = 2
_C_PB = _C_NB // _C_PH


def _agg_body(
    h2_hbm, srcg_hbm, dst_hbm, agg_hbm,
    acc_sh, src_all, dst_all, rows0, rows1, semi, sem0, sem1,
):
    c = lax.axis_index("c")
    s = lax.axis_index("s")
    gbase = c * NPAD + s * ROWS_PER_TILE
    init = pltpu.async_copy(
        h2_hbm.at[pl.ds(gbase, ROWS_PER_TILE)],
        acc_sh.at[pl.ds(s * ROWS_PER_TILE, ROWS_PER_TILE)],
        semi,
    )
    init.wait()
    plsc.subcore_barrier()

    for p in range(_C_PH):
        pltpu.async_copy(
            srcg_hbm.at[pl.ds((c * NS + s) * _C_NB + p * _C_PB, _C_PB)],
            src_all, sem0,
        ).wait()
        pltpu.async_copy(
            dst_hbm.at[pl.ds(s * _C_NB + p * _C_PB, _C_PB)], dst_all, sem1
        ).wait()

        pltpu.async_copy(h2_hbm.at[src_all.at[0]], rows0, sem0)
        pltpu.async_copy(h2_hbm.at[src_all.at[1]], rows1, sem1)

        def step(g, carry):
            j0 = 2 * g
            pltpu.make_async_copy(h2_hbm.at[src_all.at[0]], rows0, sem0).wait()
            pltpu.sync_copy(rows0, acc_sh.at[dst_all.at[j0]], add=True)
            pltpu.async_copy(h2_hbm.at[src_all.at[j0 + 2]], rows0, sem0)
            pltpu.make_async_copy(h2_hbm.at[src_all.at[0]], rows1, sem1).wait()
            pltpu.sync_copy(rows1, acc_sh.at[dst_all.at[j0 + 1]], add=True)
            pltpu.async_copy(h2_hbm.at[src_all.at[j0 + 3]], rows1, sem1)
            return carry

        lax.fori_loop(0, _C_PB // 2 - 1, step, 0)
        pltpu.make_async_copy(h2_hbm.at[src_all.at[0]], rows0, sem0).wait()
        pltpu.sync_copy(rows0, acc_sh.at[dst_all.at[_C_PB - 2]], add=True)
        pltpu.make_async_copy(h2_hbm.at[src_all.at[0]], rows1, sem1).wait()
        pltpu.sync_copy(rows1, acc_sh.at[dst_all.at[_C_PB - 1]], add=True)

    plsc.subcore_barrier()
    pltpu.async_copy(
        acc_sh.at[pl.ds(s * ROWS_PER_TILE, ROWS_PER_TILE)],
        agg_hbm.at[pl.ds(gbase, ROWS_PER_TILE)],
        semi,
    ).wait()


def _agg_call(h2, srcg2d, dst2d):
    f = pl.kernel(
        _agg_body,
        out_type=jax.ShapeDtypeStruct((NC * NPAD, DH), jnp.float32),
        mesh=_mesh(),
        scratch_types=[
            pltpu.VMEM_SHARED((NPAD, DH), jnp.float32),
            pltpu.VMEM((_C_PB, B), jnp.int32),
            pltpu.VMEM((_C_PB, B), jnp.int32),
            pltpu.VMEM((B, DH), jnp.float32),
            pltpu.VMEM((B, DH), jnp.float32),
            pltpu.SemaphoreType.DMA,
            pltpu.SemaphoreType.DMA,
            pltpu.SemaphoreType.DMA,
        ],
        name="gcn_agg_sc",
    )
    return f(h2, srcg2d, dst2d)


def _out_body(aggA_ref, aggB_ref, x_ref, dinv_ref, b_ref, out_ref):
    agg = jnp.concatenate([aggA_ref[...], aggB_ref[...]], axis=1)
    scaled = agg * dinv_ref[...] + b_ref[...][None, :]
    out_ref[...] = jnp.maximum(scaled, 0.0) + x_ref[...]


def _out_call(agg, x, dinv, b):
    R = 80
    grid = (N // R,)
    return pl.pallas_call(
        _out_body,
        grid=grid,
        in_specs=[
            pl.BlockSpec((R, DH), lambda i: (i, 0)),
            pl.BlockSpec((R, DH), lambda i: (NPAD // R + i, 0)),
            pl.BlockSpec((R, D), lambda i: (i, 0)),
            pl.BlockSpec((R, 1), lambda i: (i, 0)),
            pl.BlockSpec((D,), lambda i: (0,)),
        ],
        out_specs=pl.BlockSpec((R, D), lambda i: (i, 0)),
        out_shape=jax.ShapeDtypeStruct((N, D), jnp.float32),
        name="gcn_out_tc",
    )(agg, agg, x, dinv, b)


@jax.jit
def kernel(x, edge_index, W, b):
    src = edge_index[0].astype(jnp.int32)
    dst = edge_index[1].astype(jnp.int32)
    padfill = PADROW + jnp.arange(EPAD - E, dtype=jnp.int32) % (NPAD - N)
    src_pad = jnp.concatenate([src, padfill])
    dst_pad = jnp.concatenate([dst, padfill])
    srcg2d = jnp.concatenate([src_pad, src_pad + NPAD]).reshape(NC * (EPAD // B), B)
    dst2d = dst_pad.reshape(EPAD // B, B)
    zeros128 = jnp.zeros((ROWS_PER_TILE, DH), jnp.float32)
    ones128 = jnp.ones((B, DH), jnp.float32)

    cnt = _deg_call(dst2d, zeros128, ones128)
    h = _h_call(x, W)
    h2, dinv = _h2_call(h, cnt)
    agg = _agg_call(h2, srcg2d, dst2d)
    return _out_call(agg, x, dinv, b)


# speedup vs baseline: 1.7622x; 1.7622x over previous
import functools

import jax
import jax.numpy as jnp
from jax import lax
from jax.experimental import pallas as pl
from jax.experimental.pallas import tpu as pltpu
from jax.experimental.pallas import tpu_sc as plsc

N = 10000
E = 160000
D = 256
DH = 128
NC, NS = 2, 16
NPAD = 10240
ROWS_PER_TILE = NPAD // NS
EPAD = 163840
B = 128
PADROW = N


def _mesh():
    return plsc.VectorSubcoreMesh(
        core_axis_name="c", subcore_axis_name="s", num_cores=NC, num_subcores=NS
    )


_A_NB = EPAD // (NC * NS) // B


def _deg_body(dst_hbm, zeros_hbm, ones_hbm, cnt_hbm, deg_sh, dst_all, ones_v, sem):
    c = lax.axis_index("c")
    s = lax.axis_index("s")
    wid = s * NC + c
    pltpu.async_copy(dst_hbm.at[pl.ds(wid * _A_NB, _A_NB)], dst_all, sem).wait()
    pltpu.async_copy(ones_hbm, ones_v, sem).wait()
    pltpu.async_copy(
        zeros_hbm, deg_sh.at[pl.ds(s * ROWS_PER_TILE, ROWS_PER_TILE)], sem
    ).wait()
    plsc.subcore_barrier()

    def step(j, carry):
        pltpu.sync_copy(ones_v, deg_sh.at[dst_all.at[j]], add=True)
        return carry

    lax.fori_loop(0, _A_NB, step, 0)
    plsc.subcore_barrier()
    pltpu.async_copy(
        deg_sh.at[pl.ds(s * ROWS_PER_TILE, ROWS_PER_TILE)],
        cnt_hbm.at[pl.ds(c * NPAD + s * ROWS_PER_TILE, ROWS_PER_TILE)],
        sem,
    ).wait()


def _deg_call(dst_pad, zeros128, ones128):
    f = pl.kernel(
        _deg_body,
        out_type=jax.ShapeDtypeStruct((NC * NPAD, DH), jnp.float32),
        mesh=_mesh(),
        scratch_types=[
            pltpu.VMEM_SHARED((NPAD, DH), jnp.float32),
            pltpu.VMEM((_A_NB, B), jnp.int32),
            pltpu.VMEM((B, DH), jnp.float32),
            pltpu.SemaphoreType.DMA,
        ],
        name="gcn_deg_sc",
    )
    return f(dst_pad, zeros128, ones128)


_RB = 2000


def _h_body(x_ref, w_ref, h_ref):
    h_ref[...] = jnp.dot(
        x_ref[...], w_ref[...], preferred_element_type=jnp.float32
    )[None]


def _h_call(x, w):
    return pl.pallas_call(
        _h_body,
        grid=(NC, N // _RB),
        in_specs=[
            pl.BlockSpec((_RB, D), lambda c, i: (i, 0)),
            pl.BlockSpec((D, DH), lambda c, i: (0, c)),
        ],
        out_specs=pl.BlockSpec((1, _RB, DH), lambda c, i: (c, i, 0)),
        out_shape=jax.ShapeDtypeStruct((NC, NPAD, DH), jnp.float32),
        name="gcn_h_tc",
    )(x, w)


def _h2_body(h_ref, cnt0_ref, cnt1_ref, h2_ref, dinv_ref):
    cnt = cnt0_ref[0, :, 0] + cnt1_ref[0, :, 0] + 1.0
    dv = lax.rsqrt(cnt)
    h2_ref[...] = h_ref[...] * dv[None, :, None]
    dinv_ref[...] = dv[:, None]


def _h2_call(h, cnt3):
    return pl.pallas_call(
        _h2_body,
        grid=(NC, N // _RB),
        in_specs=[
            pl.BlockSpec((1, _RB, DH), lambda c, i: (c, i, 0)),
            pl.BlockSpec((1, _RB, DH), lambda c, i: (0, i, 0)),
            pl.BlockSpec((1, _RB, DH), lambda c, i: (1, i, 0)),
        ],
        out_specs=[
            pl.BlockSpec((1, _RB, DH), lambda c, i: (c, i, 0)),
            pl.BlockSpec((_RB, 1), lambda c, i: (i, 0)),
        ],
        out_shape=[
            jax.ShapeDtypeStruct((NC, NPAD, DH), jnp.float32),
            jax.ShapeDtypeStruct((N, 1), jnp.float32),
        ],
        name="gcn_h2_tc",
    )(h, cnt3, cnt3)


_C_NB = EPAD // NS // B
_C_PH = 2
_C_PB = _C_NB // _C_PH


def _agg_body(
    h2_hbm, srcg_hbm, dst_hbm, agg_hbm,
    acc_sh, src_all, dst_all, rows0, rows1, semi, sem0, sem1,
):
    c = lax.axis_index("c")
    s = lax.axis_index("s")
    gbase = c * NPAD + s * ROWS_PER_TILE
    init = pltpu.async_copy(
        h2_hbm.at[pl.ds(gbase, ROWS_PER_TILE)],
        acc_sh.at[pl.ds(s * ROWS_PER_TILE, ROWS_PER_TILE)],
        semi,
    )
    init.wait()
    plsc.subcore_barrier()

    for p in range(_C_PH):
        pltpu.async_copy(
            srcg_hbm.at[pl.ds((c * NS + s) * _C_NB + p * _C_PB, _C_PB)],
            src_all, sem0,
        ).wait()
        pltpu.async_copy(
            dst_hbm.at[pl.ds(s * _C_NB + p * _C_PB, _C_PB)], dst_all, sem1
        ).wait()

        pltpu.async_copy(h2_hbm.at[src_all.at[0]], rows0, sem0)
        pltpu.async_copy(h2_hbm.at[src_all.at[1]], rows1, sem1)

        def step(g, carry):
            j0 = 2 * g
            pltpu.make_async_copy(h2_hbm.at[src_all.at[0]], rows0, sem0).wait()
            pltpu.sync_copy(rows0, acc_sh.at[dst_all.at[j0]], add=True)
            pltpu.async_copy(h2_hbm.at[src_all.at[j0 + 2]], rows0, sem0)
            pltpu.make_async_copy(h2_hbm.at[src_all.at[0]], rows1, sem1).wait()
            pltpu.sync_copy(rows1, acc_sh.at[dst_all.at[j0 + 1]], add=True)
            pltpu.async_copy(h2_hbm.at[src_all.at[j0 + 3]], rows1, sem1)
            return carry

        lax.fori_loop(0, _C_PB // 2 - 1, step, 0)
        pltpu.make_async_copy(h2_hbm.at[src_all.at[0]], rows0, sem0).wait()
        pltpu.sync_copy(rows0, acc_sh.at[dst_all.at[_C_PB - 2]], add=True)
        pltpu.make_async_copy(h2_hbm.at[src_all.at[0]], rows1, sem1).wait()
        pltpu.sync_copy(rows1, acc_sh.at[dst_all.at[_C_PB - 1]], add=True)

    plsc.subcore_barrier()
    pltpu.async_copy(
        acc_sh.at[pl.ds(s * ROWS_PER_TILE, ROWS_PER_TILE)],
        agg_hbm.at[pl.ds(gbase, ROWS_PER_TILE)],
        semi,
    ).wait()


def _agg_call(h2, srcg2d, dst2d):
    f = pl.kernel(
        _agg_body,
        out_type=jax.ShapeDtypeStruct((NC * NPAD, DH), jnp.float32),
        mesh=_mesh(),
        scratch_types=[
            pltpu.VMEM_SHARED((NPAD, DH), jnp.float32),
            pltpu.VMEM((_C_PB, B), jnp.int32),
            pltpu.VMEM((_C_PB, B), jnp.int32),
            pltpu.VMEM((B, DH), jnp.float32),
            pltpu.VMEM((B, DH), jnp.float32),
            pltpu.SemaphoreType.DMA,
            pltpu.SemaphoreType.DMA,
            pltpu.SemaphoreType.DMA,
        ],
        name="gcn_agg_sc",
    )
    return f(h2, srcg2d, dst2d)


def _out_body(aggA_ref, aggB_ref, x_ref, dinv_ref, b_ref, out_ref):
    agg = jnp.concatenate([aggA_ref[...], aggB_ref[...]], axis=1)
    scaled = agg * dinv_ref[...] + b_ref[...][None, :]
    out_ref[...] = jnp.maximum(scaled, 0.0) + x_ref[...]


def _out_call(agg, x, dinv, b):
    R = 80
    grid = (N // R,)
    return pl.pallas_call(
        _out_body,
        grid=grid,
        in_specs=[
            pl.BlockSpec((R, DH), lambda i: (i, 0)),
            pl.BlockSpec((R, DH), lambda i: (NPAD // R + i, 0)),
            pl.BlockSpec((R, D), lambda i: (i, 0)),
            pl.BlockSpec((R, 1), lambda i: (i, 0)),
            pl.BlockSpec((D,), lambda i: (0,)),
        ],
        out_specs=pl.BlockSpec((R, D), lambda i: (i, 0)),
        out_shape=jax.ShapeDtypeStruct((N, D), jnp.float32),
        name="gcn_out_tc",
    )(agg, agg, x, dinv, b)


@jax.jit
def kernel(x, edge_index, W, b):
    src = edge_index[0].astype(jnp.int32)
    dst = edge_index[1].astype(jnp.int32)
    padfill = PADROW + jnp.arange(EPAD - E, dtype=jnp.int32) % (NPAD - N)
    src_pad = jnp.concatenate([src, padfill])
    dst_pad = jnp.concatenate([dst, padfill])
    srcg2d = jnp.concatenate([src_pad, src_pad + NPAD]).reshape(NC * (EPAD // B), B)
    dst2d = dst_pad.reshape(EPAD // B, B)
    zeros128 = jnp.zeros((ROWS_PER_TILE, DH), jnp.float32)
    ones128 = jnp.ones((B, DH), jnp.float32)

    cnt = _deg_call(dst2d, zeros128, ones128)
    h = _h_call(x, W)
    h2, dinv = _h2_call(h, cnt.reshape(NC, NPAD, DH))
    agg = _agg_call(h2.reshape(NC * NPAD, DH), srcg2d, dst2d)
    return _out_call(agg, x, dinv, b)
